# Initial kernel scaffold; baseline (speedup 1.0000x reference)
import jax
import jax.numpy as jnp
from jax.experimental import pallas as pl


def kernel(x, edge_index, edge_attr, W, b):
    raise NotImplementedError("write your pallas kernel here")


# speedup vs baseline: 2.6328x; 2.6328x over previous
import functools

import jax
import jax.numpy as jnp
from jax import lax
from jax.experimental import pallas as pl
from jax.experimental.pallas import tpu as pltpu
from jax.experimental.pallas import tpu_sc as plsc

N_NODES = 10000
N_EDGES = 320000
D_FEAT = 128
D_EDGE = 16
D_OUT = 128

_NC = 2
_NS = 16
_NW = _NC * _NS

_C = 128
_NCH = N_EDGES // _C
_KMAX = (_NCH + _NW - 1) // _NW
_L = 16


def _node_mm_body(x_ref, w1_ref, w2_ref, p1_ref, p2_ref):
    xv = x_ref[...]
    p1_ref[...] = jnp.dot(xv, w1_ref[...], preferred_element_type=jnp.float32)
    p2_ref[...] = jnp.dot(xv, w2_ref[...], preferred_element_type=jnp.float32)


def _edge_mm_body(ea_ref, w3_ref, b_ref, e3_ref):
    e3_ref[...] = (
        jnp.dot(ea_ref[...], w3_ref[...], preferred_element_type=jnp.float32)
        + b_ref[...]
    )


_EB = 8000


def _sc_body(p1_hbm, p2_hbm, e3_hbm, s_hbm, r_hbm, out_hbm,
             sv, rv, g1, g2, acc, sem1, sem2, sem3):
    wid = lax.axis_index("s") * _NC + lax.axis_index("c")

    def do_chunk(ci):
        base = ci * _C
        pltpu.sync_copy(s_hbm.at[pl.ds(base, _C)], sv)
        pltpu.sync_copy(r_hbm.at[pl.ds(base, _C)], rv)
        cp1 = pltpu.async_copy(p1_hbm.at[sv], g1, sem1)
        cp2 = pltpu.async_copy(p2_hbm.at[rv], g2, sem2)
        cp3 = pltpu.async_copy(e3_hbm.at[pl.ds(base, _C)], acc, sem3)
        cp1.wait()
        cp2.wait()
        cp3.wait()

        def row(i, _):
            def col(j, _):
                sl = pl.ds(j * _L, _L)
                a = acc[i, sl] + g1[i, sl] + g2[i, sl]
                acc[i, sl] = jnp.maximum(a, 0.0)
                return 0
            return lax.fori_loop(0, D_OUT // _L, col, 0)

        lax.fori_loop(0, _C, row, 0)
        pltpu.sync_copy(acc, out_hbm.at[pl.ds(base, _C)])

    def kloop(k, _):
        ci = wid + k * _NW

        @pl.when(ci < _NCH)
        def _():
            do_chunk(ci)

        return 0

    lax.fori_loop(0, _KMAX, kloop, 0)


@jax.jit
def _impl(x, s_idx, r_idx, edge_attr, W, b):
    w1 = W[0:D_FEAT]
    w2 = W[D_FEAT:2 * D_FEAT]
    w3 = W[2 * D_FEAT:]

    p1, p2 = pl.pallas_call(
        _node_mm_body,
        out_shape=(
            jax.ShapeDtypeStruct((N_NODES, D_FEAT), jnp.float32),
            jax.ShapeDtypeStruct((N_NODES, D_FEAT), jnp.float32),
        ),
    )(x, w1, w2)

    e3 = pl.pallas_call(
        _edge_mm_body,
        grid=(N_EDGES // _EB,),
        in_specs=[
            pl.BlockSpec((_EB, D_EDGE), lambda i: (i, 0)),
            pl.BlockSpec((D_EDGE, D_OUT), lambda i: (0, 0)),
            pl.BlockSpec((1, D_OUT), lambda i: (0, 0)),
        ],
        out_specs=pl.BlockSpec((_EB, D_OUT), lambda i: (i, 0)),
        out_shape=jax.ShapeDtypeStruct((N_EDGES, D_OUT), jnp.float32),
    )(edge_attr, w3, b.reshape(1, D_OUT))

    sc_fuse = functools.partial(
        pl.kernel,
        out_type=jax.ShapeDtypeStruct((N_EDGES, D_OUT), jnp.float32),
        mesh=plsc.VectorSubcoreMesh(
            core_axis_name="c", subcore_axis_name="s",
            num_cores=_NC, num_subcores=_NS,
        ),
        scratch_types=[
            pltpu.VMEM((_C,), jnp.int32),
            pltpu.VMEM((_C,), jnp.int32),
            pltpu.VMEM((_C, D_OUT), jnp.float32),
            pltpu.VMEM((_C, D_OUT), jnp.float32),
            pltpu.VMEM((_C, D_OUT), jnp.float32),
            pltpu.SemaphoreType.DMA,
            pltpu.SemaphoreType.DMA,
            pltpu.SemaphoreType.DMA,
        ],
    )(_sc_body)

    return sc_fuse(p1, p2, e3, s_idx, r_idx)


def kernel(x, edge_index, edge_attr, W, b):
    s_idx = edge_index[0].astype(jnp.int32)
    r_idx = edge_index[1].astype(jnp.int32)
    return _impl(x, s_idx, r_idx, edge_attr, W, b)


# speedup vs baseline: 2.9734x; 1.1293x over previous
import functools

import jax
import jax.numpy as jnp
from jax import lax
from jax.experimental import pallas as pl
from jax.experimental.pallas import tpu as pltpu
from jax.experimental.pallas import tpu_sc as plsc

N_NODES = 10000
N_EDGES = 320000
D_FEAT = 128
D_EDGE = 16
D_OUT = 128

_NC = 2
_NS = 16
_NW = _NC * _NS

_C = 128
_NCH = N_EDGES // _C
_NK = 80
_L = 16


def _node_mm_body(x_ref, w1_ref, w2_ref, p1_ref, p2_ref):
    xv = x_ref[...]
    p1_ref[...] = jnp.dot(xv, w1_ref[...], preferred_element_type=jnp.float32)
    p2_ref[...] = jnp.dot(xv, w2_ref[...], preferred_element_type=jnp.float32)


def _edge_mm_body(ea_ref, w3_ref, b_ref, e3_ref):
    e3_ref[...] = (
        jnp.dot(ea_ref[...], w3_ref[...], preferred_element_type=jnp.float32)
        + b_ref[...]
    )


_EB = 8000


def _sc_body(p1_hbm, p2_hbm, e3_hbm, s_hbm, r_hbm, out_hbm,
             iv, rv, g1, g2, acc,
             sem_i0, sem_i1, sem_g0, sem_g1, sem_w0, sem_w1):
    wid = lax.axis_index("s") * _NC + lax.axis_index("c")
    sem_i = (sem_i0, sem_i1)
    sem_g = (sem_g0, sem_g1)
    sem_w = (sem_w0, sem_w1)

    def chunk_base(k):
        return jnp.minimum(wid + k * _NW, _NCH - 1) * _C

    def issue_idx(k, b):
        base = chunk_base(k)
        pltpu.async_copy(s_hbm.at[pl.ds(base, _C)], iv.at[b], sem_i[b])
        pltpu.async_copy(r_hbm.at[pl.ds(base, _C)], rv.at[b], sem_i[b])

    def wait_idx(b):
        pltpu.make_async_copy(s_hbm.at[pl.ds(0, _C)], iv.at[b], sem_i[b]).wait()
        pltpu.make_async_copy(r_hbm.at[pl.ds(0, _C)], rv.at[b], sem_i[b]).wait()

    def issue_g(k, b):
        base = chunk_base(k)
        pltpu.async_copy(e3_hbm.at[pl.ds(base, _C)], acc.at[b], sem_g[b])
        pltpu.async_copy(p1_hbm.at[iv.at[b]], g1.at[b], sem_g[b])
        pltpu.async_copy(p2_hbm.at[rv.at[b]], g2.at[b], sem_g[b])

    def wait_g(b):
        pltpu.make_async_copy(e3_hbm.at[pl.ds(0, _C)], acc.at[b], sem_g[b]).wait()
        pltpu.make_async_copy(p1_hbm.at[iv.at[b]], g1.at[b], sem_g[b]).wait()
        pltpu.make_async_copy(p2_hbm.at[rv.at[b]], g2.at[b], sem_g[b]).wait()

    def issue_wb(k, b):
        base = chunk_base(k)
        pltpu.async_copy(acc.at[b], out_hbm.at[pl.ds(base, _C)], sem_w[b])

    def wait_wb(b):
        pltpu.make_async_copy(
            acc.at[b], out_hbm.at[pl.ds(0, _C)], sem_w[b]).wait()

    def compute(b):
        accb, g1b, g2b = acc.at[b], g1.at[b], g2.at[b]

        @pl.loop(0, _C, unroll=2)
        def _(i):
            for j in range(D_OUT // _L):
                sl = pl.ds(j * _L, _L)
                a = accb[i, sl] + g1b[i, sl] + g2b[i, sl]
                accb[i, sl] = jnp.maximum(a, 0.0)

    issue_idx(0, 0)
    issue_idx(1, 1)
    wait_idx(0)
    issue_g(0, 0)

    @pl.loop(0, _NK, step=2)
    def _(kk):
        for d in range(2):
            k = kk + d
            b = d
            bn = 1 - d

            wait_g(b)

            @pl.when(k + 1 < _NK)
            def _():
                wait_idx(bn)

                @pl.when(k >= 1)
                def _():
                    wait_wb(bn)

                issue_g(k + 1, bn)

            @pl.when(k + 2 < _NK)
            def _():
                issue_idx(k + 2, b)

            compute(b)
            issue_wb(k, b)

    wait_wb(0)
    wait_wb(1)


@jax.jit
def _impl(x, s_idx, r_idx, edge_attr, W, b):
    w1 = W[0:D_FEAT]
    w2 = W[D_FEAT:2 * D_FEAT]
    w3 = W[2 * D_FEAT:]

    p1, p2 = pl.pallas_call(
        _node_mm_body,
        out_shape=(
            jax.ShapeDtypeStruct((N_NODES, D_FEAT), jnp.float32),
            jax.ShapeDtypeStruct((N_NODES, D_FEAT), jnp.float32),
        ),
    )(x, w1, w2)

    e3 = pl.pallas_call(
        _edge_mm_body,
        grid=(N_EDGES // _EB,),
        in_specs=[
            pl.BlockSpec((_EB, D_EDGE), lambda i: (i, 0)),
            pl.BlockSpec((D_EDGE, D_OUT), lambda i: (0, 0)),
            pl.BlockSpec((1, D_OUT), lambda i: (0, 0)),
        ],
        out_specs=pl.BlockSpec((_EB, D_OUT), lambda i: (i, 0)),
        out_shape=jax.ShapeDtypeStruct((N_EDGES, D_OUT), jnp.float32),
    )(edge_attr, w3, b.reshape(1, D_OUT))

    sc_fuse = functools.partial(
        pl.kernel,
        out_type=jax.ShapeDtypeStruct((N_EDGES, D_OUT), jnp.float32),
        mesh=plsc.VectorSubcoreMesh(
            core_axis_name="c", subcore_axis_name="s",
            num_cores=_NC, num_subcores=_NS,
        ),
        scratch_types=[
            pltpu.VMEM((2, _C), jnp.int32),
            pltpu.VMEM((2, _C), jnp.int32),
            pltpu.VMEM((2, _C, D_OUT), jnp.float32),
            pltpu.VMEM((2, _C, D_OUT), jnp.float32),
            pltpu.VMEM((2, _C, D_OUT), jnp.float32),
            pltpu.SemaphoreType.DMA,
            pltpu.SemaphoreType.DMA,
            pltpu.SemaphoreType.DMA,
            pltpu.SemaphoreType.DMA,
            pltpu.SemaphoreType.DMA,
            pltpu.SemaphoreType.DMA,
        ],
    )(_sc_body)

    return sc_fuse(p1, p2, e3, s_idx, r_idx)


def kernel(x, edge_index, edge_attr, W, b):
    s_idx = edge_index[0].astype(jnp.int32)
    r_idx = edge_index[1].astype(jnp.int32)
    return _impl(x, s_idx, r_idx, edge_attr, W, b)
